# baseline (speedup 1.0000x reference)
import functools

import jax
import jax.numpy as jnp
from jax import lax
from jax.experimental import pallas as pl
from jax.experimental.pallas import tpu as pltpu
from jax.experimental.pallas import tpu_sc as plsc

B, L, D = 4096, 200, 128
NL = 10
LP = 208
CH = LP // 2
NC, NS, LANES = 2, 16, 16
NW = NC * NS
RPW = B // NW
NVR = D // LANES


NBUF = 4
NCH = RPW * 2


def _make_bag():
    mesh = plsc.VectorSubcoreMesh(core_axis_name="c", subcore_axis_name="s")

    @functools.partial(
        pl.kernel,
        mesh=mesh,
        out_type=jax.ShapeDtypeStruct((B, D), jnp.float32),
        scratch_types=[
            pltpu.VMEM((RPW * LP,), jnp.int32),
            pltpu.VMEM((NBUF, CH, D), jnp.float32),
            pltpu.VMEM((RPW, D), jnp.float32),
            pltpu.SemaphoreType.DMA,
            pltpu.SemaphoreType.DMA,
            pltpu.SemaphoreType.DMA,
            pltpu.SemaphoreType.DMA,
        ],
    )
    def bag(x_hbm, emb_hbm, out_hbm, idx_v, rows_v, z_v, s0, s1, s2, s3):
        sems = (s0, s1, s2, s3)
        wid = lax.axis_index("s") * NC + lax.axis_index("c")
        base = wid * RPW
        pltpu.sync_copy(
            x_hbm.at[pl.ds(pl.multiple_of(base * LP, 8), RPW * LP)], idx_v)

        def copy_desc(c, slot):
            off = pl.multiple_of(c * CH, 8)
            return pltpu.make_async_copy(
                emb_hbm.at[idx_v.at[pl.ds(off, CH)]], rows_v.at[slot],
                sems[slot])

        for k in range(NBUF - 1):
            copy_desc(k, k).start()

        def reduce_chunk(slot, accs):
            def red(r, accs):
                return tuple(
                    a + rows_v[slot, 2 * r, pl.ds(j * LANES, LANES)]
                    + rows_v[slot, 2 * r + 1, pl.ds(j * LANES, LANES)]
                    for j, a in enumerate(accs))
            return lax.fori_loop(0, CH // 2, red, accs)

        zero = tuple(jnp.zeros((LANES,), jnp.float32) for _ in range(NVR))

        def group_body(g, carry):
            for k in range(NBUF):
                c = 4 * g + k
                f = c + NBUF - 1
                fslot = (k + NBUF - 1) % NBUF

                @pl.when(f < NCH)
                def _():
                    copy_desc(f, fslot).start()

                copy_desc(c, k).wait()
                if k % 2 == 1:
                    accs = zero
                    row = 2 * g + k // 2
                    for j in range(NVR):
                        z_v[row, pl.ds(j * LANES, LANES)] = accs[j]
            return carry

        lax.fori_loop(0, NCH // NBUF, group_body, 0)
        pltpu.sync_copy(z_v, out_hbm.at[pl.ds(base, RPW)])

    return bag


_bag = _make_bag()


def _head_body(s_ref, x_ref, wp_ref, bp_ref, wf_ref, bf_ref, o_ref):
    cnt = jnp.sum((x_ref[...] != 0).astype(jnp.float32), axis=1, keepdims=True)
    z = s_ref[...] / jnp.maximum(cnt, 1.0)
    h = lax.dot_general(z, wp_ref[...], (((1,), (1,)), ((), ())),
                        preferred_element_type=jnp.float32)
    h = jnp.maximum(h + bp_ref[...], 0.0)
    o = lax.dot_general(h, wf_ref[...], (((1,), (1,)), ((), ())),
                        preferred_element_type=jnp.float32)
    o_ref[...] = o + bf_ref[...]


BT = 512


_head = pl.pallas_call(
    _head_body,
    grid=(B // BT,),
    in_specs=[
        pl.BlockSpec((BT, D), lambda i: (i, 0)),
        pl.BlockSpec((BT, L), lambda i: (i, 0)),
        pl.BlockSpec((D, D), lambda i: (0, 0)),
        pl.BlockSpec((1, D), lambda i: (0, 0)),
        pl.BlockSpec((NL, D), lambda i: (0, 0)),
        pl.BlockSpec((1, NL), lambda i: (0, 0)),
    ],
    out_specs=pl.BlockSpec((BT, NL), lambda i: (i, 0)),
    out_shape=jax.ShapeDtypeStruct((B, NL), jnp.float32),
)


def kernel(x, emb, Wp, bp, Wf, bf):
    x_pad = jnp.pad(x, ((0, 0), (0, LP - L))).reshape(B * LP)
    sums = _bag(x_pad, emb)
    return _head(sums, x, Wp, bp.reshape(1, D), Wf, bf.reshape(1, NL))


# speedup vs baseline: 1.1818x; 1.1818x over previous
import functools

import jax
import jax.numpy as jnp
from jax import lax
from jax.experimental import pallas as pl
from jax.experimental.pallas import tpu as pltpu
from jax.experimental.pallas import tpu_sc as plsc

B, L, D = 4096, 200, 128
VOC = 100000
NL = 10
LP = 208
CH = LP // 2
NC, NS, LANES = 2, 16, 16
NW = NC * NS
RPW = B // NW
NVR = D // LANES


NBUF = 4
NCH = RPW * 2


def _make_bag():
    mesh = plsc.VectorSubcoreMesh(core_axis_name="c", subcore_axis_name="s")

    @functools.partial(
        pl.kernel,
        mesh=mesh,
        compiler_params=pltpu.CompilerParams(use_tc_tiling_on_sc=False),
        out_type=jax.ShapeDtypeStruct((B, D), jnp.float32),
        scratch_types=[
            pltpu.VMEM((RPW * LP,), jnp.int32),
            pltpu.VMEM((NBUF, CH, D // 2), jnp.int32),
            pltpu.VMEM((RPW, D), jnp.float32),
            pltpu.SemaphoreType.DMA,
            pltpu.SemaphoreType.DMA,
            pltpu.SemaphoreType.DMA,
            pltpu.SemaphoreType.DMA,
        ],
    )
    def bag(x_hbm, emb_hbm, out_hbm, idx_v, rows_v, z_v, s0, s1, s2, s3):
        sems = (s0, s1, s2, s3)
        wid = lax.axis_index("s") * NC + lax.axis_index("c")
        base = wid * RPW
        pltpu.sync_copy(
            x_hbm.at[pl.ds(pl.multiple_of(base * LP, 8), RPW * LP)], idx_v)

        def copy_desc(c, slot):
            off = pl.multiple_of(c * CH, 8)
            return pltpu.make_async_copy(
                emb_hbm.at[idx_v.at[pl.ds(off, CH)]], rows_v.at[slot],
                sems[slot])

        for k in range(NBUF - 1):
            copy_desc(k, k).start()

        def reduce_chunk(slot, accs):
            def red(r, accs):
                return tuple(
                    a + rows_v[slot, 2 * r, pl.ds(j * LANES, LANES)]
                    + rows_v[slot, 2 * r + 1, pl.ds(j * LANES, LANES)]
                    for j, a in enumerate(accs))
            return lax.fori_loop(0, CH // 2, red, accs)

        zero = tuple(jnp.zeros((LANES,), jnp.float32) for _ in range(NVR))

        def group_body(g, carry):
            for k in range(NBUF):
                c = 4 * g + k
                f = c + NBUF - 1
                fslot = (k + NBUF - 1) % NBUF

                @pl.when(f < NCH)
                def _():
                    copy_desc(f, fslot).start()

                copy_desc(c, k).wait()
                if k % 2 == 1:
                    accs = zero
                    row = 2 * g + k // 2
                    for j in range(NVR):
                        z_v[row, pl.ds(j * LANES, LANES)] = accs[j]
            return carry

        lax.fori_loop(0, NCH // NBUF, group_body, 0)
        pltpu.sync_copy(z_v, out_hbm.at[pl.ds(base, RPW)])

    return bag


_bag = _make_bag()


def _head_body(s_ref, x_ref, wp_ref, bp_ref, wf_ref, bf_ref, o_ref):
    cnt = jnp.sum((x_ref[...] != 0).astype(jnp.float32), axis=1, keepdims=True)
    z = s_ref[...] / jnp.maximum(cnt, 1.0)
    h = lax.dot_general(z, wp_ref[...], (((1,), (1,)), ((), ())),
                        preferred_element_type=jnp.float32)
    h = jnp.maximum(h + bp_ref[...], 0.0)
    o = lax.dot_general(h, wf_ref[...], (((1,), (1,)), ((), ())),
                        preferred_element_type=jnp.float32)
    o_ref[...] = o + bf_ref[...]


BT = 512


_head = pl.pallas_call(
    _head_body,
    grid=(B // BT,),
    in_specs=[
        pl.BlockSpec((BT, D), lambda i: (i, 0)),
        pl.BlockSpec((BT, L), lambda i: (i, 0)),
        pl.BlockSpec((D, D), lambda i: (0, 0)),
        pl.BlockSpec((1, D), lambda i: (0, 0)),
        pl.BlockSpec((NL, D), lambda i: (0, 0)),
        pl.BlockSpec((1, NL), lambda i: (0, 0)),
    ],
    out_specs=pl.BlockSpec((BT, NL), lambda i: (i, 0)),
    out_shape=jax.ShapeDtypeStruct((B, NL), jnp.float32),
)


def kernel(x, emb, Wp, bp, Wf, bf):
    x_pad = jnp.pad(x, ((0, 0), (0, LP - L))).reshape(B * LP)
    emb_h = jax.lax.bitcast_convert_type(
        emb.astype(jnp.bfloat16).reshape(VOC, D // 2, 2), jnp.int32)
    sums = _bag(x_pad, emb_h)
    return _head(sums, x, Wp, bp.reshape(1, D), Wf, bf.reshape(1, NL))


# speedup vs baseline: 2.0572x; 1.7408x over previous
import functools

import jax
import jax.numpy as jnp
import numpy as np
from jax import lax
from jax.experimental import pallas as pl
from jax.experimental.pallas import tpu as pltpu
from jax.experimental.pallas import tpu_sc as plsc

B, L, D = 4096, 200, 128
VOC = 100000
NL = 10
LP = 208
NC, NS, LANES = 2, 16, 16
NW = NC * NS
RPW = B // NW
CPR = LP // LANES

BLKLG = 13
BLK = 1 << BLKLG
NBLK = 13
VPAD = NBLK * BLK
HW = D // 2
DUMMY = BLK
FILL = BLK // NS

CAP0 = 4352
CAPK = 3264
BUCKW = CAP0 + (NBLK - 1) * CAPK

CHK = 128
NB = 3
HRPW = RPW // 2
GRP = 8
GPC = CHK // GRP
NJ = HW // LANES

SM_CNT = 0
SM_CUR = 4 * RPW


def _kbase(k):
    return k * CAPK + min(k, 1) * (CAP0 - CAPK) if isinstance(k, int) else (
        k * CAPK + lax.min(k, 1) * (CAP0 - CAPK))


def _make_bag():
    mesh = plsc.VectorSubcoreMesh(core_axis_name="c", subcore_axis_name="s")

    @functools.partial(
        pl.kernel,
        mesh=mesh,
        compiler_params=pltpu.CompilerParams(
            use_tc_tiling_on_sc=False, needs_layout_passes=False),
        out_type=jax.ShapeDtypeStruct((B, D), jnp.float32),
        scratch_types=[
            pltpu.VMEM((HRPW * LP,), jnp.int32),
            pltpu.VMEM((BUCKW,), jnp.int32),
            pltpu.VMEM((NB * CHK, HW), jnp.int32),
            pltpu.VMEM((RPW, D), jnp.float32),
            pltpu.VMEM((LANES,), jnp.int32),
            pltpu.VMEM_SHARED((BLK + 8, HW), jnp.int32),
            pltpu.SMEM((SM_CUR + NBLK,), jnp.int32),
            pltpu.SemaphoreType.DMA,
            pltpu.SemaphoreType.DMA,
            pltpu.SemaphoreType.DMA,
            pltpu.SemaphoreType.DMA,
        ],
    )
    def bag(x_hbm, emb_hbm, out_hbm, idx_v, buck_v, ring_v, z_v, cur_v,
            sh_v, sm, s0, s1, s2, s3):
        sems = (s0, s1, s2, s3)
        sid = lax.axis_index("s")
        wid = sid * NC + lax.axis_index("c")
        base = wid * RPW

        zeros16 = jnp.zeros((LANES,), jnp.float32)
        dummy16 = jnp.full((LANES,), DUMMY, jnp.int32)

        def zbody(r, carry):
            for q in range(D // LANES):
                z_v[r, pl.ds(q * LANES, LANES)] = zeros16
            return carry
        lax.fori_loop(0, RPW, zbody, 0)

        @pl.when(sid == 0)
        def _():
            zi = jnp.zeros((LANES,), jnp.int32)
            for r in range(8):
                for jj in range(NJ):
                    ring_v[r, pl.ds(jj * LANES, LANES)] = zi
            pltpu.sync_copy(ring_v.at[pl.ds(0, 8)], sh_v.at[pl.ds(BLK, 8)])

        iota = lax.broadcasted_iota(jnp.int32, (LANES,), 0)
        basev = jnp.where(iota == 0, 0,
                          CAP0 + (iota - 1) * CAPK).astype(jnp.int32)
        gd = lax.GatherDimensionNumbers(offset_dims=(),
                                        collapsed_slice_dims=(0,),
                                        start_index_map=(0,))

        def take16(vec, idx):
            return lax.gather(vec, idx[:, None], gd, (1,),
                              mode=lax.GatherScatterMode.PROMISE_IN_BOUNDS)

        def fillbody(i, carry):
            buck_v[pl.ds(pl.multiple_of(i * LANES, 8), LANES)] = dummy16
            return carry
        lax.fori_loop(0, BUCKW // LANES, fillbody, 0)
        cur_v[pl.ds(0, LANES)] = jnp.zeros((LANES,), jnp.int32)

        def p1body_half(h, r, carry):
            prevv = cur_v[pl.ds(0, LANES)]

            def chunk_body(i, carry):
                off = pl.multiple_of(r * LP + i * LANES, 8)
                v = idx_v[pl.ds(off, LANES)]
                blk = jax.lax.shift_right_logical(v, BLKLG)
                loc = v & jnp.int32(BLK - 1)
                def dbody(d, rc):
                    rank, anyf = rc
                    bm = iota >= d
                    eb = (take16(blk, jnp.maximum(iota - d, 0)) == blk) & bm
                    rank = rank + eb.astype(jnp.int32)
                    fm = iota <= (LANES - 1) - d
                    ef = (take16(blk, jnp.minimum(iota + d, LANES - 1))
                          == blk) & fm
                    return (rank, anyf | ef)

                rank, anyf = lax.fori_loop(
                    1, LANES, dbody,
                    (jnp.zeros((LANES,), jnp.int32),
                     jnp.zeros((LANES,), jnp.bool_)))
                curg = take16(cur_v[pl.ds(0, LANES)], blk)
                baseg = take16(basev, blk)
                plsc.store_scatter(buck_v, [baseg + curg + rank], loc)
                plsc.store_scatter(cur_v, [blk], curg + rank + 1,
                                   mask=jnp.logical_not(anyf))
                return carry

            lax.fori_loop(0, CPR, chunk_body, 0)
            curv = cur_v[pl.ds(0, LANES)]
            curv = prevv + ((curv - prevv + (GRP - 1))
                            & ~jnp.int32(GRP - 1))
            cur_v[pl.ds(0, LANES)] = curv
            ring_v[h * HRPW + r, pl.ds(0, LANES)] = curv - prevv
            return carry

        for h in range(2):
            pltpu.sync_copy(
                x_hbm.at[pl.ds(pl.multiple_of((base + h * HRPW) * LP, 8),
                               HRPW * LP)], idx_v)
            lax.fori_loop(0, HRPW, functools.partial(p1body_half, h), 0)

        totv = ((cur_v[pl.ds(0, LANES)] + (CHK - 1))
                & ~jnp.int32(CHK - 1))
        for k in range(NBLK):
            sm[SM_CUR + k] = totv[k]

        def cntbody(r, carry):
            cv = ring_v[r, pl.ds(0, LANES)]
            for j in range(4):
                w = jnp.int32(0)
                for t in range(4):
                    k = 4 * j + t
                    if k < NBLK:
                        w = w | (cv[k] << (8 * t))
                sm[r * 4 + j] = w
            return carry
        lax.fori_loop(0, RPW, cntbody, 0)

        def desc(k, c, slot, sh_ref):
            koff = pl.multiple_of(_kbase(k) + c * CHK, 8)
            soff = pl.multiple_of(slot * CHK, 8)
            return pltpu.make_async_copy(
                sh_ref.at[buck_v.at[pl.ds(koff, CHK)]],
                ring_v.at[pl.ds(soff, CHK)],
                s0)

        def pass_body(k, carry):
            plsc.subcore_barrier()
            pltpu.sync_copy(
                emb_hbm.at[pl.ds(pl.multiple_of(k * BLK + sid * FILL, 8),
                                 FILL)],
                sh_v.at[pl.ds(pl.multiple_of(sid * FILL, 8), FILL)])
            plsc.subcore_barrier()

            ntok = sm[SM_CUR + k]
            nchunks = jax.lax.shift_right_logical(ntok, 7)
            for cc in range(NB - 1):
                @pl.when(cc < nchunks)
                def _():
                    desc(k, cc, jnp.int32(cc), sh_v).start()

            def row_body(r, st):
                c, slot, gic = st
                wj = sm[r * 4 + jax.lax.shift_right_logical(k, 2)]
                cnt = jax.lax.shift_right_logical(
                    wj, 8 * (k & 3)) & jnp.int32(255)
                ngrp = jax.lax.shift_right_logical(cnt, 3)
                accs = [z_v[r, pl.ds(q * LANES, LANES)]
                        for q in range(D // LANES)]

                def grp_body(g, st2):
                    c, slot, gic, *accs = st2

                    @pl.when(gic == 0)
                    def _():
                        desc(k, c, slot, sh_v).wait()

                        @pl.when(c + NB - 1 < nchunks)
                        def _():
                            fslot = slot + jnp.int32(NB - 1)
                            fslot = fslot - jnp.int32(NB) * (
                                fslot >= jnp.int32(NB)).astype(jnp.int32)
                            desc(k, c + NB - 1, fslot, sh_v).start()

                    rbase = slot * CHK + gic * GRP
                    for u in range(GRP):
                        for jj in range(NJ):
                            w = ring_v[rbase + u, pl.ds(jj * LANES, LANES)]
                            bf = plsc.bitcast(w, jnp.bfloat16)
                            a, b = plsc.unpack(
                                bf, format=plsc.PackFormat.INTERLEAVED)
                            accs[2 * jj] = accs[2 * jj] + a
                            accs[2 * jj + 1] = accs[2 * jj + 1] + b
                    gic = gic + 1
                    wrap = (gic == GPC).astype(jnp.int32)
                    nslot = slot + wrap
                    nslot = nslot - jnp.int32(NB) * (
                        nslot >= jnp.int32(NB)).astype(jnp.int32)
                    return (c + wrap, nslot, gic * (1 - wrap)) + tuple(accs)

                out = lax.fori_loop(0, ngrp, grp_body,
                                    (c, slot, gic) + tuple(accs))
                c, slot, gic = out[0], out[1], out[2]
                accs = out[3:]
                for q in range(D // LANES):
                    z_v[r, pl.ds(q * LANES, LANES)] = accs[q]
                return (c, slot, gic)

            lax.fori_loop(0, RPW, row_body,
                          (jnp.int32(0), jnp.int32(0), jnp.int32(0)))
            return carry

        lax.fori_loop(0, NBLK, pass_body, 0)
        pltpu.sync_copy(z_v, out_hbm.at[pl.ds(pl.multiple_of(base, 8), RPW)])

    return bag


_bag = _make_bag()


def _head_body(s_ref, x_ref, wp_ref, bp_ref, wf_ref, bf_ref, o_ref):
    cnt = jnp.sum((x_ref[...] != 0).astype(jnp.float32), axis=1, keepdims=True)
    z = s_ref[...] / jnp.maximum(cnt, 1.0)
    h = lax.dot_general(z, wp_ref[...], (((1,), (1,)), ((), ())),
                        preferred_element_type=jnp.float32)
    h = jnp.maximum(h + bp_ref[...], 0.0)
    o = lax.dot_general(h, wf_ref[...], (((1,), (1,)), ((), ())),
                        preferred_element_type=jnp.float32)
    o_ref[...] = o + bf_ref[...]


BT = 512


_head = pl.pallas_call(
    _head_body,
    grid=(B // BT,),
    in_specs=[
        pl.BlockSpec((BT, D), lambda i: (i, 0)),
        pl.BlockSpec((BT, L), lambda i: (i, 0)),
        pl.BlockSpec((D, D), lambda i: (0, 0)),
        pl.BlockSpec((1, D), lambda i: (0, 0)),
        pl.BlockSpec((NL, D), lambda i: (0, 0)),
        pl.BlockSpec((1, NL), lambda i: (0, 0)),
    ],
    out_specs=pl.BlockSpec((BT, NL), lambda i: (i, 0)),
    out_shape=jax.ShapeDtypeStruct((B, NL), jnp.float32),
)

_PERM = np.array([32 * (c // 32) + 2 * (c % 16) + (c % 32) // 16
                  for c in range(D)])


def kernel(x, emb, Wp, bp, Wf, bf):
    x_pad = jnp.pad(x, ((0, 0), (0, LP - L))).reshape(B * LP)
    emb_pad = jnp.pad(emb.astype(jnp.bfloat16), ((0, VPAD - VOC), (0, 0)))
    emb_h = lax.bitcast_convert_type(
        emb_pad.reshape(VPAD, HW, 2), jnp.int32)
    sums = _bag(x_pad, emb_h)
    return _head(sums, x, Wp[:, _PERM], bp.reshape(1, D), Wf,
                 bf.reshape(1, NL))


# speedup vs baseline: 2.0896x; 1.0158x over previous
import functools

import jax
import jax.numpy as jnp
import numpy as np
from jax import lax
from jax.experimental import pallas as pl
from jax.experimental.pallas import tpu as pltpu
from jax.experimental.pallas import tpu_sc as plsc

B, L, D = 4096, 200, 128
VOC = 100000
NL = 10
LP = 208
NC, NS, LANES = 2, 16, 16
NW = NC * NS
RPW = B // NW
CPR = LP // LANES

BLKLG = 13
BLK = 1 << BLKLG
NBLK = 13
VPAD = NBLK * BLK
HW = D // 2
DUMMY = BLK
FILL = BLK // NS

CAP0 = 4352
CAPK = 3264
BUCKW = CAP0 + (NBLK - 1) * CAPK

CHK = 128
NB = 3
HRPW = RPW // 2
GRP = 8
GPC = CHK // GRP
NJ = HW // LANES

SM_CNT = 0
SM_CUR = 4 * RPW


def _kbase(k):
    return k * CAPK + min(k, 1) * (CAP0 - CAPK) if isinstance(k, int) else (
        k * CAPK + lax.min(k, 1) * (CAP0 - CAPK))


def _make_bag():
    mesh = plsc.VectorSubcoreMesh(core_axis_name="c", subcore_axis_name="s")

    @functools.partial(
        pl.kernel,
        mesh=mesh,
        compiler_params=pltpu.CompilerParams(
            use_tc_tiling_on_sc=False, needs_layout_passes=False),
        out_type=jax.ShapeDtypeStruct((B, D), jnp.float32),
        scratch_types=[
            pltpu.VMEM((HRPW * LP,), jnp.int32),
            pltpu.VMEM((BUCKW,), jnp.int32),
            pltpu.VMEM((NB * CHK, HW), jnp.int32),
            pltpu.VMEM((RPW, D), jnp.float32),
            pltpu.VMEM((LANES,), jnp.int32),
            pltpu.VMEM_SHARED((BLK + 8, HW), jnp.int32),
            pltpu.SMEM((SM_CUR + NBLK,), jnp.int32),
            pltpu.SemaphoreType.DMA,
            pltpu.SemaphoreType.DMA,
            pltpu.SemaphoreType.DMA,
            pltpu.SemaphoreType.DMA,
        ],
    )
    def bag(x_hbm, emb_hbm, out_hbm, idx_v, buck_v, ring_v, z_v, cur_v,
            sh_v, sm, s0, s1, s2, s3):
        sems = (s0, s1, s2, s3)
        sid = lax.axis_index("s")
        wid = sid * NC + lax.axis_index("c")
        base = wid * RPW

        zeros16 = jnp.zeros((LANES,), jnp.float32)
        dummy16 = jnp.full((LANES,), DUMMY, jnp.int32)

        def zbody(r, carry):
            for q in range(D // LANES):
                z_v[r, pl.ds(q * LANES, LANES)] = zeros16
            return carry
        lax.fori_loop(0, RPW, zbody, 0)

        @pl.when(sid == 0)
        def _():
            zi = jnp.zeros((LANES,), jnp.int32)
            for r in range(8):
                for jj in range(NJ):
                    ring_v[r, pl.ds(jj * LANES, LANES)] = zi
            pltpu.sync_copy(ring_v.at[pl.ds(0, 8)], sh_v.at[pl.ds(BLK, 8)])

        iota = lax.broadcasted_iota(jnp.int32, (LANES,), 0)
        basev = jnp.where(iota == 0, 0,
                          CAP0 + (iota - 1) * CAPK).astype(jnp.int32)
        gd = lax.GatherDimensionNumbers(offset_dims=(),
                                        collapsed_slice_dims=(0,),
                                        start_index_map=(0,))

        def take16(vec, idx):
            return lax.gather(vec, idx[:, None], gd, (1,),
                              mode=lax.GatherScatterMode.PROMISE_IN_BOUNDS)

        def fillbody(i, carry):
            buck_v[pl.ds(pl.multiple_of(i * LANES, 8), LANES)] = dummy16
            return carry
        lax.fori_loop(0, BUCKW // LANES, fillbody, 0)
        cur_v[pl.ds(0, LANES)] = jnp.zeros((LANES,), jnp.int32)

        def p1body_half(h, r, carry):
            prevv = cur_v[pl.ds(0, LANES)]

            def chunk_body(i, carry):
                off = pl.multiple_of(r * LP + i * LANES, 8)
                v = idx_v[pl.ds(off, LANES)]
                blk = jax.lax.shift_right_logical(v, BLKLG)
                loc = v & jnp.int32(BLK - 1)
                def dbody(g, rc):
                    rank, anyf = rc
                    for dd in range(3):
                        d = 3 * g + (1 + dd)
                        bm = iota >= d
                        eb = (take16(blk, jnp.maximum(iota - d, 0))
                              == blk) & bm
                        rank = rank + eb.astype(jnp.int32)
                        fm = iota <= (LANES - 1) - d
                        ef = (take16(blk, jnp.minimum(iota + d, LANES - 1))
                              == blk) & fm
                        anyf = anyf | ef
                    return (rank, anyf)

                rank, anyf = lax.fori_loop(
                    0, 5, dbody,
                    (jnp.zeros((LANES,), jnp.int32),
                     jnp.zeros((LANES,), jnp.bool_)))
                curg = take16(cur_v[pl.ds(0, LANES)], blk)
                baseg = take16(basev, blk)
                plsc.store_scatter(buck_v, [baseg + curg + rank], loc)
                plsc.store_scatter(cur_v, [blk], curg + rank + 1,
                                   mask=jnp.logical_not(anyf))
                return carry

            lax.fori_loop(0, CPR, chunk_body, 0)
            curv = cur_v[pl.ds(0, LANES)]
            curv = prevv + ((curv - prevv + (GRP - 1))
                            & ~jnp.int32(GRP - 1))
            cur_v[pl.ds(0, LANES)] = curv
            ring_v[h * HRPW + r, pl.ds(0, LANES)] = curv - prevv
            return carry

        for h in range(2):
            pltpu.sync_copy(
                x_hbm.at[pl.ds(pl.multiple_of((base + h * HRPW) * LP, 8),
                               HRPW * LP)], idx_v)
            lax.fori_loop(0, HRPW, functools.partial(p1body_half, h), 0)

        totv = ((cur_v[pl.ds(0, LANES)] + (CHK - 1))
                & ~jnp.int32(CHK - 1))
        for k in range(NBLK):
            sm[SM_CUR + k] = totv[k]

        def cntbody(r, carry):
            cv = ring_v[r, pl.ds(0, LANES)]
            for j in range(4):
                w = jnp.int32(0)
                for t in range(4):
                    k = 4 * j + t
                    if k < NBLK:
                        w = w | (cv[k] << (8 * t))
                sm[r * 4 + j] = w
            return carry
        lax.fori_loop(0, RPW, cntbody, 0)

        def desc(k, c, slot, sh_ref):
            koff = pl.multiple_of(_kbase(k) + c * CHK, 8)
            soff = pl.multiple_of(slot * CHK, 8)
            return pltpu.make_async_copy(
                sh_ref.at[buck_v.at[pl.ds(koff, CHK)]],
                ring_v.at[pl.ds(soff, CHK)],
                s0)

        def pass_body(k, carry):
            plsc.subcore_barrier()
            pltpu.sync_copy(
                emb_hbm.at[pl.ds(pl.multiple_of(k * BLK + sid * FILL, 8),
                                 FILL)],
                sh_v.at[pl.ds(pl.multiple_of(sid * FILL, 8), FILL)])
            plsc.subcore_barrier()

            ntok = sm[SM_CUR + k]
            nchunks = jax.lax.shift_right_logical(ntok, 7)
            for cc in range(NB - 1):
                @pl.when(cc < nchunks)
                def _():
                    desc(k, cc, jnp.int32(cc), sh_v).start()

            def row_body(r, st):
                c, slot, gic = st
                wj = sm[r * 4 + jax.lax.shift_right_logical(k, 2)]
                cnt = jax.lax.shift_right_logical(
                    wj, 8 * (k & 3)) & jnp.int32(255)
                ngrp = jax.lax.shift_right_logical(cnt, 3)
                accs = [z_v[r, pl.ds(q * LANES, LANES)]
                        for q in range(D // LANES)]

                def grp_body(g, st2):
                    c, slot, gic, *accs = st2

                    @pl.when(gic == 0)
                    def _():
                        desc(k, c, slot, sh_v).wait()

                        @pl.when(c + NB - 1 < nchunks)
                        def _():
                            fslot = slot + jnp.int32(NB - 1)
                            fslot = fslot - jnp.int32(NB) * (
                                fslot >= jnp.int32(NB)).astype(jnp.int32)
                            desc(k, c + NB - 1, fslot, sh_v).start()

                    rbase = slot * CHK + gic * GRP
                    for u in range(GRP):
                        for jj in range(NJ):
                            w = ring_v[rbase + u, pl.ds(jj * LANES, LANES)]
                            bf = plsc.bitcast(w, jnp.bfloat16)
                            a, b = plsc.unpack(
                                bf, format=plsc.PackFormat.INTERLEAVED)
                            accs[2 * jj] = accs[2 * jj] + a
                            accs[2 * jj + 1] = accs[2 * jj + 1] + b
                    gic = gic + 1
                    wrap = (gic == GPC).astype(jnp.int32)
                    nslot = slot + wrap
                    nslot = nslot - jnp.int32(NB) * (
                        nslot >= jnp.int32(NB)).astype(jnp.int32)
                    return (c + wrap, nslot, gic * (1 - wrap)) + tuple(accs)

                out = lax.fori_loop(0, ngrp, grp_body,
                                    (c, slot, gic) + tuple(accs))
                c, slot, gic = out[0], out[1], out[2]
                accs = out[3:]
                for q in range(D // LANES):
                    z_v[r, pl.ds(q * LANES, LANES)] = accs[q]
                return (c, slot, gic)

            lax.fori_loop(0, RPW, row_body,
                          (jnp.int32(0), jnp.int32(0), jnp.int32(0)))
            return carry

        lax.fori_loop(0, NBLK, pass_body, 0)
        pltpu.sync_copy(z_v, out_hbm.at[pl.ds(pl.multiple_of(base, 8), RPW)])

    return bag


_bag = _make_bag()


def _head_body(s_ref, x_ref, wp_ref, bp_ref, wf_ref, bf_ref, o_ref):
    cnt = jnp.sum((x_ref[...] != 0).astype(jnp.float32), axis=1, keepdims=True)
    z = s_ref[...] / jnp.maximum(cnt, 1.0)
    h = lax.dot_general(z, wp_ref[...], (((1,), (1,)), ((), ())),
                        preferred_element_type=jnp.float32)
    h = jnp.maximum(h + bp_ref[...], 0.0)
    o = lax.dot_general(h, wf_ref[...], (((1,), (1,)), ((), ())),
                        preferred_element_type=jnp.float32)
    o_ref[...] = o + bf_ref[...]


BT = 512


_head = pl.pallas_call(
    _head_body,
    grid=(B // BT,),
    in_specs=[
        pl.BlockSpec((BT, D), lambda i: (i, 0)),
        pl.BlockSpec((BT, L), lambda i: (i, 0)),
        pl.BlockSpec((D, D), lambda i: (0, 0)),
        pl.BlockSpec((1, D), lambda i: (0, 0)),
        pl.BlockSpec((NL, D), lambda i: (0, 0)),
        pl.BlockSpec((1, NL), lambda i: (0, 0)),
    ],
    out_specs=pl.BlockSpec((BT, NL), lambda i: (i, 0)),
    out_shape=jax.ShapeDtypeStruct((B, NL), jnp.float32),
)

_PERM = np.array([32 * (c // 32) + 2 * (c % 16) + (c % 32) // 16
                  for c in range(D)])


def kernel(x, emb, Wp, bp, Wf, bf):
    x_pad = jnp.pad(x, ((0, 0), (0, LP - L))).reshape(B * LP)
    emb_pad = jnp.pad(emb.astype(jnp.bfloat16), ((0, VPAD - VOC), (0, 0)))
    emb_h = lax.bitcast_convert_type(
        emb_pad.reshape(VPAD, HW, 2), jnp.int32)
    sums = _bag(x_pad, emb_h)
    return _head(sums, x, Wp[:, _PERM], bp.reshape(1, D), Wf,
                 bf.reshape(1, NL))


# speedup vs baseline: 2.2119x; 1.0585x over previous
import functools

import jax
import jax.numpy as jnp
import numpy as np
from jax import lax
from jax.experimental import pallas as pl
from jax.experimental.pallas import tpu as pltpu
from jax.experimental.pallas import tpu_sc as plsc

B, L, D = 4096, 200, 128
VOC = 100000
NL = 10
LP = 208
NC, NS, LANES = 2, 16, 16
NW = NC * NS
RPW = B // NW
CPR = LP // LANES

BLKLG = 13
BLK = 1 << BLKLG
NBLK = 13
VPAD = NBLK * BLK
HW = D // 2
DUMMY = BLK
FILL = BLK // NS

CAP0 = 4352
CAPK = 3264
BUCKW = CAP0 + (NBLK - 1) * CAPK

CHK = 128
NB = 3
HRPW = RPW // 2
GRP = 8
GPC = CHK // GRP
NJ = HW // LANES

SM_CNT = 0
SM_CUR = 4 * RPW


def _kbase(k):
    return k * CAPK + min(k, 1) * (CAP0 - CAPK) if isinstance(k, int) else (
        k * CAPK + lax.min(k, 1) * (CAP0 - CAPK))


def _make_bag():
    mesh = plsc.VectorSubcoreMesh(core_axis_name="c", subcore_axis_name="s")

    @functools.partial(
        pl.kernel,
        mesh=mesh,
        compiler_params=pltpu.CompilerParams(
            use_tc_tiling_on_sc=False, needs_layout_passes=False),
        out_type=jax.ShapeDtypeStruct((B, D), jnp.float32),
        scratch_types=[
            pltpu.VMEM((HRPW * LP,), jnp.int32),
            pltpu.VMEM((BUCKW,), jnp.int32),
            pltpu.VMEM((NB * CHK, HW), jnp.int32),
            pltpu.VMEM((RPW, D), jnp.float32),
            pltpu.VMEM_SHARED((BLK + 8, HW), jnp.int32),
            pltpu.SMEM((SM_CUR + NBLK,), jnp.int32),
            pltpu.SemaphoreType.DMA,
            pltpu.SemaphoreType.DMA,
            pltpu.SemaphoreType.DMA,
            pltpu.SemaphoreType.DMA,
        ],
    )
    def bag(x_hbm, emb_hbm, out_hbm, idx_v, buck_v, ring_v, z_v, sh_v, sm,
            s0, s1, s2, s3):
        sems = (s0, s1, s2, s3)
        sid = lax.axis_index("s")
        wid = sid * NC + lax.axis_index("c")
        base = wid * RPW

        zeros16 = jnp.zeros((LANES,), jnp.float32)
        dummy16 = jnp.full((LANES,), DUMMY, jnp.int32)

        def zbody(r, carry):
            for q in range(D // LANES):
                z_v[r, pl.ds(q * LANES, LANES)] = zeros16
            return carry
        lax.fori_loop(0, RPW, zbody, 0)

        @pl.when(sid == 0)
        def _():
            zi = jnp.zeros((LANES,), jnp.int32)
            for r in range(8):
                for jj in range(NJ):
                    ring_v[r, pl.ds(jj * LANES, LANES)] = zi
            pltpu.sync_copy(ring_v.at[pl.ds(0, 8)], sh_v.at[pl.ds(BLK, 8)])

        def p1body_half(h, r, cur):
            prev = cur
            for i in range(CPR):
                off = pl.multiple_of(r * LP + i * LANES, 8)
                v = idx_v[pl.ds(off, LANES)]
                blk = jax.lax.shift_right_logical(v, BLKLG)
                loc = v & jnp.int32(BLK - 1)
                ms = [blk == k for k in range(NBLK)]
                pcs = [plsc.all_reduce_population_count(ms[k])
                       for k in range(NBLK)]
                for k in range(NBLK):
                    plsc.store_compressed(
                        buck_v.at[pl.ds(_kbase(k) + cur[k], LANES)], loc,
                        mask=ms[k])
                cur = tuple(cur[k] + pcs[k][0] for k in range(NBLK))
            ncur = []
            for k in range(NBLK):
                buck_v[pl.ds(_kbase(k) + cur[k], LANES)] = dummy16
                ncur.append((cur[k] + (GRP - 1)) & ~jnp.int32(GRP - 1))
            cur = tuple(ncur)
            for j in range(4):
                w = jnp.int32(0)
                for t in range(4):
                    k = 4 * j + t
                    if k < NBLK:
                        w = w | ((cur[k] - prev[k]) << (8 * t))
                sm[(h * HRPW + r) * 4 + j] = w
            return cur

        cur = tuple(jnp.int32(0) for _ in range(NBLK))
        for h in range(2):
            pltpu.sync_copy(
                x_hbm.at[pl.ds(pl.multiple_of((base + h * HRPW) * LP, 8),
                               HRPW * LP)], idx_v)
            cur = lax.fori_loop(0, HRPW, functools.partial(p1body_half, h),
                                cur)
        for k in range(NBLK):
            for t in range(GPC // 2):
                buck_v[pl.ds(_kbase(k) + cur[k] + t * LANES, LANES)] = dummy16
            sm[SM_CUR + k] = (cur[k] + (CHK - 1)) & ~jnp.int32(CHK - 1)

        def desc(k, c, slot, sh_ref):
            koff = pl.multiple_of(_kbase(k) + c * CHK, 8)
            soff = pl.multiple_of(slot * CHK, 8)
            return pltpu.make_async_copy(
                sh_ref.at[buck_v.at[pl.ds(koff, CHK)]],
                ring_v.at[pl.ds(soff, CHK)],
                s0)

        def pass_body(k, carry):
            plsc.subcore_barrier()
            pltpu.sync_copy(
                emb_hbm.at[pl.ds(pl.multiple_of(k * BLK + sid * FILL, 8),
                                 FILL)],
                sh_v.at[pl.ds(pl.multiple_of(sid * FILL, 8), FILL)])
            plsc.subcore_barrier()

            ntok = sm[SM_CUR + k]
            nchunks = jax.lax.shift_right_logical(ntok, 7)
            for cc in range(NB - 1):
                @pl.when(cc < nchunks)
                def _():
                    desc(k, cc, jnp.int32(cc), sh_v).start()

            def row_body(r, st):
                c, slot, gic = st
                wj = sm[r * 4 + jax.lax.shift_right_logical(k, 2)]
                cnt = jax.lax.shift_right_logical(
                    wj, 8 * (k & 3)) & jnp.int32(255)
                ngrp = jax.lax.shift_right_logical(cnt, 3)
                accs = [z_v[r, pl.ds(q * LANES, LANES)]
                        for q in range(D // LANES)]

                def grp_body(g, st2):
                    c, slot, gic, *accs = st2

                    @pl.when(gic == 0)
                    def _():
                        desc(k, c, slot, sh_v).wait()

                        @pl.when(c + NB - 1 < nchunks)
                        def _():
                            fslot = slot + jnp.int32(NB - 1)
                            fslot = fslot - jnp.int32(NB) * (
                                fslot >= jnp.int32(NB)).astype(jnp.int32)
                            desc(k, c + NB - 1, fslot, sh_v).start()

                    rbase = slot * CHK + gic * GRP
                    for u in range(GRP):
                        for jj in range(NJ):
                            w = ring_v[rbase + u, pl.ds(jj * LANES, LANES)]
                            bf = plsc.bitcast(w, jnp.bfloat16)
                            a, b = plsc.unpack(
                                bf, format=plsc.PackFormat.INTERLEAVED)
                            accs[2 * jj] = accs[2 * jj] + a
                            accs[2 * jj + 1] = accs[2 * jj + 1] + b
                    gic = gic + 1
                    wrap = (gic == GPC).astype(jnp.int32)
                    nslot = slot + wrap
                    nslot = nslot - jnp.int32(NB) * (
                        nslot >= jnp.int32(NB)).astype(jnp.int32)
                    return (c + wrap, nslot, gic * (1 - wrap)) + tuple(accs)

                out = lax.fori_loop(0, ngrp, grp_body,
                                    (c, slot, gic) + tuple(accs))
                c, slot, gic = out[0], out[1], out[2]
                accs = out[3:]
                for q in range(D // LANES):
                    z_v[r, pl.ds(q * LANES, LANES)] = accs[q]
                return (c, slot, gic)

            lax.fori_loop(0, RPW, row_body,
                          (jnp.int32(0), jnp.int32(0), jnp.int32(0)))
            return carry

        lax.fori_loop(0, NBLK, pass_body, 0)
        pltpu.sync_copy(z_v, out_hbm.at[pl.ds(pl.multiple_of(base, 8), RPW)])

    return bag


_bag = _make_bag()


def _head_body(s_ref, x_ref, wp_ref, bp_ref, wf_ref, bf_ref, o_ref):
    cnt = jnp.sum((x_ref[...] != 0).astype(jnp.float32), axis=1, keepdims=True)
    z = s_ref[...] / jnp.maximum(cnt, 1.0)
    h = lax.dot_general(z, wp_ref[...], (((1,), (1,)), ((), ())),
                        preferred_element_type=jnp.float32)
    h = jnp.maximum(h + bp_ref[...], 0.0)
    o = lax.dot_general(h, wf_ref[...], (((1,), (1,)), ((), ())),
                        preferred_element_type=jnp.float32)
    o_ref[...] = o + bf_ref[...]


BT = 512


_head = pl.pallas_call(
    _head_body,
    grid=(B // BT,),
    in_specs=[
        pl.BlockSpec((BT, D), lambda i: (i, 0)),
        pl.BlockSpec((BT, L), lambda i: (i, 0)),
        pl.BlockSpec((D, D), lambda i: (0, 0)),
        pl.BlockSpec((1, D), lambda i: (0, 0)),
        pl.BlockSpec((NL, D), lambda i: (0, 0)),
        pl.BlockSpec((1, NL), lambda i: (0, 0)),
    ],
    out_specs=pl.BlockSpec((BT, NL), lambda i: (i, 0)),
    out_shape=jax.ShapeDtypeStruct((B, NL), jnp.float32),
)

_PERM = np.array([32 * (c // 32) + 2 * (c % 16) + (c % 32) // 16
                  for c in range(D)])


def kernel(x, emb, Wp, bp, Wf, bf):
    x_pad = jnp.pad(x, ((0, 0), (0, LP - L))).reshape(B * LP)
    emb_pad = jnp.pad(emb.astype(jnp.bfloat16), ((0, VPAD - VOC), (0, 0)))
    emb_h = lax.bitcast_convert_type(
        emb_pad.reshape(VPAD, HW, 2), jnp.int32)
    sums = _bag(x_pad, emb_h)
    return _head(sums, x, Wp[:, _PERM], bp.reshape(1, D), Wf,
                 bf.reshape(1, NL))
